# baseline (speedup 1.0000x reference)
import functools

import jax
import jax.numpy as jnp
from jax import lax
from jax.experimental import pallas as pl
from jax.experimental.pallas import tpu as pltpu
from jax.experimental.pallas import tpu_sc as plsc

N = 10000
NP = 10240
E = 160000
NSUB = 16
EW = 10240
JH = 40
BR = 64
RPW = BR // NSUB
NBLK = NP // BR
NBLK_SC = NBLK // 2
ZW = 5120
CROWS = 160
BD = 1024
G = NP // BD


def _sc_build_p1(rows_hbm, cols_hbm, zeros_hbm, p1_hbm,
                 rows_v, cols_v, vals_v, coff_v, cval_v, cnts_v, curs_v,
                 rst_v, ones_v, zero_v, deg_sh, block_sh, sem_b):
    c = lax.axis_index("c")
    s = lax.axis_index("s")
    i32 = jnp.int32
    lane = lax.iota(jnp.int32, 16)

    pltpu.sync_copy(zeros_hbm, zero_v)
    for i in range(8):
        ones_v[pl.ds(i * 16, 16)] = jnp.full((16,), 1.0, jnp.float32)

    pltpu.sync_copy(zero_v.at[pl.ds(0, NP // NSUB)],
                    deg_sh.at[pl.ds(s * (NP // NSUB), NP // NSUB)])

    def _zero_cnts(p, _):
        cnts_v[p, :] = jnp.zeros((16,), jnp.int32)
        return i32(0)
    lax.fori_loop(i32(0), i32(NBLK_SC), _zero_cnts, i32(0))
    plsc.subcore_barrier()

    for h in range(2):
        pltpu.sync_copy(rows_hbm.at[s, jnp.int32(h)], rows_v)

        def _deg_fire(j, _):
            pltpu.async_copy(ones_v, deg_sh.at[rows_v.at[j]], sem_b,
                             add=True)
            return i32(0)
        lax.fori_loop(i32(0), i32(JH), _deg_fire, i32(0))

        def _hist(j, _):
            for i in range(8):
                sl = pl.ds(i * 16, 16)
                rv = rows_v[j, sl]
                p = lax.shift_right_logical(rv, jnp.int32(6)) - c * NBLK_SC
                m = (p >= 0) & (p < NBLK_SC)
                pm = jnp.where(m, p, i32(0))
                plsc.addupdate_scatter(cnts_v, (pm, lane),
                                       jnp.ones((16,), jnp.int32), mask=m)
            return i32(0)
        lax.fori_loop(i32(0), i32(JH), _hist, i32(0))

        def _deg_drain(j, _):
            pltpu.make_async_copy(ones_v, deg_sh.at[rows_v.at[j]],
                                  sem_b).wait()
            return i32(0)
        lax.fori_loop(i32(0), i32(JH), _deg_drain, i32(0))
    plsc.subcore_barrier()

    def _starts_p(p, row_acc):
        rst_v[p] = row_acc
        x = cnts_v[p, :]
        cs = plsc.cumsum(x)
        curs_v[p, :] = row_acc * 128 + cs - x
        n_p = jnp.sum(x, dtype=jnp.int32)
        return row_acc + lax.shift_right_logical(n_p + 127, jnp.int32(7))
    total_rows = lax.fori_loop(i32(0), i32(NBLK_SC), _starts_p, i32(0))
    rst_v[NBLK_SC] = total_rows

    def _zero_comp(r, _):
        for i in range(8):
            sl = pl.ds(i * 16, 16)
            coff_v[r, sl] = jnp.zeros((16,), jnp.int32)
            cval_v[r, sl] = jnp.zeros((16,), jnp.float32)
        return i32(0)
    lax.fori_loop(i32(0), i32(CROWS), _zero_comp, i32(0))

    for h in range(2):
        pltpu.sync_copy(rows_hbm.at[s, jnp.int32(h)], rows_v)
        pltpu.sync_copy(cols_hbm.at[s, jnp.int32(h)], cols_v)

        def _val_fire(j, _):
            pltpu.async_copy(deg_sh.at[rows_v.at[j]], vals_v.at[j], sem_b)
            return i32(0)
        lax.fori_loop(i32(0), i32(JH), _val_fire, i32(0))

        def _val_drain(j, _):
            pltpu.make_async_copy(deg_sh.at[rows_v.at[j]], vals_v.at[j],
                                  sem_b).wait()
            return i32(0)
        lax.fori_loop(i32(0), i32(JH), _val_drain, i32(0))

        def _place(j, _):
            for i in range(8):
                sl = pl.ds(i * 16, 16)
                rv = rows_v[j, sl]
                cv = cols_v[j, sl]
                p = lax.shift_right_logical(rv, jnp.int32(6)) - c * NBLK_SC
                m = (p >= 0) & (p < NBLK_SC)
                pm = jnp.where(m, p, i32(0))
                cur = plsc.load_gather(curs_v, (pm, lane), mask=m)
                plsc.store_scatter(curs_v, (pm, lane), cur + 1, mask=m)
                vv = 1.0 / vals_v[j, sl]
                off = (rv & i32(63)) * NP + cv
                rhi = lax.shift_right_logical(cur, jnp.int32(7))
                rlo = cur & i32(127)
                plsc.store_scatter(coff_v, (rhi, rlo), off, mask=m)
                plsc.store_scatter(cval_v, (rhi, rlo), vv, mask=m)
            return i32(0)
        lax.fori_loop(i32(0), i32(JH), _place, i32(0))

    my_off = s * (RPW * NP)
    nz = RPW * NP // ZW

    def _pass_body(p, _):
        r0 = (c * NBLK_SC + p) * BR
        for z in range(nz):
            pltpu.async_copy(zero_v,
                             block_sh.at[pl.ds(my_off + z * ZW, ZW)], sem_b)
        for z in range(nz):
            pltpu.make_async_copy(zero_v,
                                  block_sh.at[pl.ds(my_off + z * ZW, ZW)],
                                  sem_b).wait()
        plsc.subcore_barrier()

        ra = rst_v[p]
        rb = rst_v[p + 1]

        def _scat_fire(r, _):
            pltpu.async_copy(cval_v.at[r], block_sh.at[coff_v.at[r]],
                             sem_b, add=True)
            return i32(0)
        lax.fori_loop(ra, rb, _scat_fire, i32(0))

        def _scat_drain(r, _):
            pltpu.make_async_copy(cval_v.at[r], block_sh.at[coff_v.at[r]],
                                  sem_b).wait()
            return i32(0)
        lax.fori_loop(ra, rb, _scat_drain, i32(0))
        plsc.subcore_barrier()

        dst = r0 * NP + my_off
        pltpu.sync_copy(block_sh.at[pl.ds(my_off, RPW * NP)],
                        p1_hbm.at[pl.ds(dst, RPW * NP)])
        return i32(0)
    lax.fori_loop(i32(0), i32(NBLK_SC), _pass_body, i32(0))


def _build_p1(rows, cols, zeros_small):
    kern = pl.kernel(
        _sc_build_p1,
        out_type=jax.ShapeDtypeStruct((NP * NP,), jnp.float32),
        mesh=plsc.VectorSubcoreMesh(core_axis_name="c", subcore_axis_name="s"),
        compiler_params=pltpu.CompilerParams(needs_layout_passes=False),
        scratch_types=[
            pltpu.VMEM((JH, 128), jnp.int32),
            pltpu.VMEM((JH, 128), jnp.int32),
            pltpu.VMEM((JH, 128), jnp.float32),
            pltpu.VMEM((CROWS, 128), jnp.int32),
            pltpu.VMEM((CROWS, 128), jnp.float32),
            pltpu.VMEM((NBLK_SC, 16), jnp.int32),
            pltpu.VMEM((NBLK_SC, 16), jnp.int32),
            pltpu.SMEM((NBLK_SC + 8,), jnp.int32),
            pltpu.VMEM((128,), jnp.float32),
            pltpu.VMEM((ZW,), jnp.float32),
            pltpu.VMEM_SHARED((NP,), jnp.float32),
            pltpu.VMEM_SHARED((BR * NP,), jnp.float32),
            pltpu.SemaphoreType.DMA,
        ],
    )
    return kern(rows, cols, zeros_small)


def _z():
    return jnp.int32(0)


def _mm_body(a_ref, b_ref, o_ref):
    k = pl.program_id(2)
    part = jnp.dot(a_ref[...], b_ref[...], preferred_element_type=jnp.float32)

    @pl.when(k == 0)
    def _():
        o_ref[...] = part

    @pl.when(k != 0)
    def _():
        o_ref[...] = o_ref[...] + part


def _matmul(a, b, bm=2048, bn=1024, bk=1280):
    return pl.pallas_call(
        _mm_body,
        grid=(NP // bm, NP // bn, NP // bk),
        in_specs=[
            pl.BlockSpec((bm, bk), lambda i, j, k: (i, k)),
            pl.BlockSpec((bk, bn), lambda i, j, k: (k, j)),
        ],
        out_specs=pl.BlockSpec((bm, bn), lambda i, j, k: (i, j)),
        out_shape=jax.ShapeDtypeStruct((NP, NP), jnp.float32),
        compiler_params=pltpu.CompilerParams(
            dimension_semantics=("parallel", "parallel", "arbitrary")),
    )(a, b)


def _diag_body(p4r, p1c, p2c, p3c, p4c, out, acc):
    i = pl.program_id(0)
    k = pl.program_id(1)

    @pl.when(k == 0)
    def _():
        acc[...] = jnp.zeros_like(acc)

    eye = (lax.broadcasted_iota(jnp.int32, (BD, BD), 0)
           == lax.broadcasted_iota(jnp.int32, (BD, BD), 1)).astype(jnp.float32)
    a = p4r[...]
    c1, c2, c3, c4 = p1c[...], p2c[...], p3c[...], p4c[...]
    d5 = jnp.sum(jnp.dot(a, c1, preferred_element_type=jnp.float32) * eye, 1)
    d6 = jnp.sum(jnp.dot(a, c2, preferred_element_type=jnp.float32) * eye, 1)
    d7 = jnp.sum(jnp.dot(a, c3, preferred_element_type=jnp.float32) * eye, 1)
    d8 = jnp.sum(jnp.dot(a, c4, preferred_element_type=jnp.float32) * eye, 1)
    ondiag = jnp.where(k == i, jnp.float32(1.0), jnp.float32(0.0))
    g1 = jnp.sum(c1 * eye, 0) * ondiag
    g2 = jnp.sum(c2 * eye, 0) * ondiag
    g3 = jnp.sum(c3 * eye, 0) * ondiag
    g4 = jnp.sum(c4 * eye, 0) * ondiag
    acc[...] = acc[...] + jnp.stack([g1, g2, g3, g4, d5, d6, d7, d8], 0)

    @pl.when(k == G - 1)
    def _():
        out[...] = acc[...]


def _diag_all(p1, p2, p3, p4):
    return pl.pallas_call(
        _diag_body,
        grid=(G, G),
        in_specs=[
            pl.BlockSpec((BD, BD), lambda i, k: (i, k)),
            pl.BlockSpec((BD, BD), lambda i, k: (k, i)),
            pl.BlockSpec((BD, BD), lambda i, k: (k, i)),
            pl.BlockSpec((BD, BD), lambda i, k: (k, i)),
            pl.BlockSpec((BD, BD), lambda i, k: (k, i)),
        ],
        out_specs=pl.BlockSpec((8, BD), lambda i, k: (_z(), i)),
        out_shape=jax.ShapeDtypeStruct((8, NP), jnp.float32),
        scratch_shapes=[pltpu.VMEM((8, BD), jnp.float32)],
    )(p4, p1, p2, p3, p4)


def kernel(x, edge_index):
    row = edge_index[0].astype(jnp.int32)
    col = edge_index[1].astype(jnp.int32)
    pad = NSUB * EW - E
    row = jnp.concatenate([row, jnp.full((pad,), NP - 1, jnp.int32)])
    col = jnp.concatenate([col, jnp.full((pad,), NP - 1, jnp.int32)])
    rows = row.reshape(NSUB, 2, JH, 128)
    cols = col.reshape(NSUB, 2, JH, 128)
    zeros_small = jnp.zeros((ZW,), jnp.float32)

    p1 = _build_p1(rows, cols, zeros_small).reshape(NP, NP)
    p2 = _matmul(p1, p1)
    p3 = _matmul(p1, p2)
    p4 = _matmul(p2, p2)
    out8 = _diag_all(p1, p2, p3, p4)
    return out8.T[:N].astype(x.dtype)


# speedup vs baseline: 1.0124x; 1.0124x over previous
import functools

import jax
import jax.numpy as jnp
from jax import lax
from jax.experimental import pallas as pl
from jax.experimental.pallas import tpu as pltpu
from jax.experimental.pallas import tpu_sc as plsc

N = 10000
NP = 10240
E = 160000
NSUB = 16
EW = 10240
JH = 40
BR = 64
RPW = BR // NSUB
NBLK = NP // BR
NBLK_SC = NBLK // 2
ZW = 5120
CROWS = 160
BD = 512
G = NP // BD


def _sc_build_p1(rows_hbm, cols_hbm, zeros_hbm, p1_hbm,
                 rows_v, cols_v, vals_v, coff_v, cval_v, cnts_v, curs_v,
                 rst_v, ones_v, zero_v, deg_sh, block_sh, sem_b):
    c = lax.axis_index("c")
    s = lax.axis_index("s")
    i32 = jnp.int32
    lane = lax.iota(jnp.int32, 16)

    pltpu.sync_copy(zeros_hbm, zero_v)
    for i in range(8):
        ones_v[pl.ds(i * 16, 16)] = jnp.full((16,), 1.0, jnp.float32)

    pltpu.sync_copy(zero_v.at[pl.ds(0, NP // NSUB)],
                    deg_sh.at[pl.ds(s * (NP // NSUB), NP // NSUB)])

    def _zero_cnts(p, _):
        cnts_v[p, :] = jnp.zeros((16,), jnp.int32)
        return i32(0)
    lax.fori_loop(i32(0), i32(NBLK_SC), _zero_cnts, i32(0))
    plsc.subcore_barrier()

    for h in range(2):
        pltpu.sync_copy(rows_hbm.at[s, jnp.int32(h)], rows_v)

        def _deg_fire(j, _):
            pltpu.async_copy(ones_v, deg_sh.at[rows_v.at[j]], sem_b,
                             add=True)
            return i32(0)
        lax.fori_loop(i32(0), i32(JH), _deg_fire, i32(0))

        def _hist(j, _):
            for i in range(8):
                sl = pl.ds(i * 16, 16)
                rv = rows_v[j, sl]
                p = lax.shift_right_logical(rv, jnp.int32(6)) - c * NBLK_SC
                m = (p >= 0) & (p < NBLK_SC)
                pm = jnp.where(m, p, i32(0))
                plsc.addupdate_scatter(cnts_v, (pm, lane),
                                       jnp.ones((16,), jnp.int32), mask=m)
            return i32(0)
        lax.fori_loop(i32(0), i32(JH), _hist, i32(0))

        def _deg_drain(j, _):
            pltpu.make_async_copy(ones_v, deg_sh.at[rows_v.at[j]],
                                  sem_b).wait()
            return i32(0)
        lax.fori_loop(i32(0), i32(JH), _deg_drain, i32(0))
    plsc.subcore_barrier()

    def _starts_p(p, row_acc):
        rst_v[p] = row_acc
        x = cnts_v[p, :]
        cs = plsc.cumsum(x)
        curs_v[p, :] = row_acc * 128 + cs - x
        n_p = jnp.sum(x, dtype=jnp.int32)
        return row_acc + lax.shift_right_logical(n_p + 127, jnp.int32(7))
    total_rows = lax.fori_loop(i32(0), i32(NBLK_SC), _starts_p, i32(0))
    rst_v[NBLK_SC] = total_rows

    def _zero_comp(r, _):
        for i in range(8):
            sl = pl.ds(i * 16, 16)
            coff_v[r, sl] = jnp.zeros((16,), jnp.int32)
            cval_v[r, sl] = jnp.zeros((16,), jnp.float32)
        return i32(0)
    lax.fori_loop(i32(0), i32(CROWS), _zero_comp, i32(0))

    for h in range(2):
        pltpu.sync_copy(rows_hbm.at[s, jnp.int32(h)], rows_v)
        pltpu.sync_copy(cols_hbm.at[s, jnp.int32(h)], cols_v)

        def _val_fire(j, _):
            pltpu.async_copy(deg_sh.at[rows_v.at[j]], vals_v.at[j], sem_b)
            return i32(0)
        lax.fori_loop(i32(0), i32(JH), _val_fire, i32(0))

        def _val_drain(j, _):
            pltpu.make_async_copy(deg_sh.at[rows_v.at[j]], vals_v.at[j],
                                  sem_b).wait()
            return i32(0)
        lax.fori_loop(i32(0), i32(JH), _val_drain, i32(0))

        def _place(j, _):
            for i in range(8):
                sl = pl.ds(i * 16, 16)
                rv = rows_v[j, sl]
                cv = cols_v[j, sl]
                p = lax.shift_right_logical(rv, jnp.int32(6)) - c * NBLK_SC
                m = (p >= 0) & (p < NBLK_SC)
                pm = jnp.where(m, p, i32(0))
                cur = plsc.load_gather(curs_v, (pm, lane), mask=m)
                plsc.store_scatter(curs_v, (pm, lane), cur + 1, mask=m)
                vv = 1.0 / vals_v[j, sl]
                off = (rv & i32(63)) * NP + cv
                rhi = lax.shift_right_logical(cur, jnp.int32(7))
                rlo = cur & i32(127)
                plsc.store_scatter(coff_v, (rhi, rlo), off, mask=m)
                plsc.store_scatter(cval_v, (rhi, rlo), vv, mask=m)
            return i32(0)
        lax.fori_loop(i32(0), i32(JH), _place, i32(0))

    my_off = s * (RPW * NP)
    nz = RPW * NP // ZW

    def _pass_body(p, _):
        r0 = (c * NBLK_SC + p) * BR
        for z in range(nz):
            pltpu.async_copy(zero_v,
                             block_sh.at[pl.ds(my_off + z * ZW, ZW)], sem_b)
        for z in range(nz):
            pltpu.make_async_copy(zero_v,
                                  block_sh.at[pl.ds(my_off + z * ZW, ZW)],
                                  sem_b).wait()
        plsc.subcore_barrier()

        ra = rst_v[p]
        rb = rst_v[p + 1]

        def _scat_fire(r, _):
            pltpu.async_copy(cval_v.at[r], block_sh.at[coff_v.at[r]],
                             sem_b, add=True)
            return i32(0)
        lax.fori_loop(ra, rb, _scat_fire, i32(0))

        def _scat_drain(r, _):
            pltpu.make_async_copy(cval_v.at[r], block_sh.at[coff_v.at[r]],
                                  sem_b).wait()
            return i32(0)
        lax.fori_loop(ra, rb, _scat_drain, i32(0))
        plsc.subcore_barrier()

        dst = r0 * NP + my_off
        pltpu.sync_copy(block_sh.at[pl.ds(my_off, RPW * NP)],
                        p1_hbm.at[pl.ds(dst, RPW * NP)])
        return i32(0)
    lax.fori_loop(i32(0), i32(NBLK_SC), _pass_body, i32(0))


def _build_p1(rows, cols, zeros_small):
    kern = pl.kernel(
        _sc_build_p1,
        out_type=jax.ShapeDtypeStruct((NP * NP,), jnp.float32),
        mesh=plsc.VectorSubcoreMesh(core_axis_name="c", subcore_axis_name="s"),
        compiler_params=pltpu.CompilerParams(needs_layout_passes=False),
        scratch_types=[
            pltpu.VMEM((JH, 128), jnp.int32),
            pltpu.VMEM((JH, 128), jnp.int32),
            pltpu.VMEM((JH, 128), jnp.float32),
            pltpu.VMEM((CROWS, 128), jnp.int32),
            pltpu.VMEM((CROWS, 128), jnp.float32),
            pltpu.VMEM((NBLK_SC, 16), jnp.int32),
            pltpu.VMEM((NBLK_SC, 16), jnp.int32),
            pltpu.SMEM((NBLK_SC + 8,), jnp.int32),
            pltpu.VMEM((128,), jnp.float32),
            pltpu.VMEM((ZW,), jnp.float32),
            pltpu.VMEM_SHARED((NP,), jnp.float32),
            pltpu.VMEM_SHARED((BR * NP,), jnp.float32),
            pltpu.SemaphoreType.DMA,
        ],
    )
    return kern(rows, cols, zeros_small)


def _z():
    return jnp.int32(0)


def _mm_body(a_ref, b_ref, o_ref):
    k = pl.program_id(2)
    part = jnp.dot(a_ref[...], b_ref[...], preferred_element_type=jnp.float32)

    @pl.when(k == 0)
    def _():
        o_ref[...] = part

    @pl.when(k != 0)
    def _():
        o_ref[...] = o_ref[...] + part


def _matmul(a, b, bm=2048, bn=1024, bk=1280):
    return pl.pallas_call(
        _mm_body,
        grid=(NP // bm, NP // bn, NP // bk),
        in_specs=[
            pl.BlockSpec((bm, bk), lambda i, j, k: (i, k)),
            pl.BlockSpec((bk, bn), lambda i, j, k: (k, j)),
        ],
        out_specs=pl.BlockSpec((bm, bn), lambda i, j, k: (i, j)),
        out_shape=jax.ShapeDtypeStruct((NP, NP), jnp.float32),
        compiler_params=pltpu.CompilerParams(
            dimension_semantics=("parallel", "parallel", "arbitrary")),
    )(a, b)


def _diag_body(p4r, p1c, p2c, p3c, p4c, out, acc):
    i = pl.program_id(0)
    k = pl.program_id(1)

    @pl.when(k == 0)
    def _():
        acc[...] = jnp.zeros_like(acc)

    eye = (lax.broadcasted_iota(jnp.int32, (BD, BD), 0)
           == lax.broadcasted_iota(jnp.int32, (BD, BD), 1)).astype(jnp.float32)
    a = p4r[...]
    c1, c2, c3, c4 = p1c[...], p2c[...], p3c[...], p4c[...]
    d5 = jnp.sum(jnp.dot(a, c1, preferred_element_type=jnp.float32) * eye, 1)
    d6 = jnp.sum(jnp.dot(a, c2, preferred_element_type=jnp.float32) * eye, 1)
    d7 = jnp.sum(jnp.dot(a, c3, preferred_element_type=jnp.float32) * eye, 1)
    d8 = jnp.sum(jnp.dot(a, c4, preferred_element_type=jnp.float32) * eye, 1)
    ondiag = jnp.where(k == i, jnp.float32(1.0), jnp.float32(0.0))
    g1 = jnp.sum(c1 * eye, 0) * ondiag
    g2 = jnp.sum(c2 * eye, 0) * ondiag
    g3 = jnp.sum(c3 * eye, 0) * ondiag
    g4 = jnp.sum(c4 * eye, 0) * ondiag
    acc[...] = acc[...] + jnp.stack([g1, g2, g3, g4, d5, d6, d7, d8], 0)

    @pl.when(k == G - 1)
    def _():
        out[...] = acc[...]


def _diag_all(p1, p2, p3, p4):
    return pl.pallas_call(
        _diag_body,
        grid=(G, G),
        in_specs=[
            pl.BlockSpec((BD, BD), lambda i, k: (i, k)),
            pl.BlockSpec((BD, BD), lambda i, k: (k, i)),
            pl.BlockSpec((BD, BD), lambda i, k: (k, i)),
            pl.BlockSpec((BD, BD), lambda i, k: (k, i)),
            pl.BlockSpec((BD, BD), lambda i, k: (k, i)),
        ],
        out_specs=pl.BlockSpec((8, BD), lambda i, k: (_z(), i)),
        out_shape=jax.ShapeDtypeStruct((8, NP), jnp.float32),
        scratch_shapes=[pltpu.VMEM((8, BD), jnp.float32)],
    )(p4, p1, p2, p3, p4)


def kernel(x, edge_index):
    row = edge_index[0].astype(jnp.int32)
    col = edge_index[1].astype(jnp.int32)
    pad = NSUB * EW - E
    row = jnp.concatenate([row, jnp.full((pad,), NP - 1, jnp.int32)])
    col = jnp.concatenate([col, jnp.full((pad,), NP - 1, jnp.int32)])
    rows = row.reshape(NSUB, 2, JH, 128)
    cols = col.reshape(NSUB, 2, JH, 128)
    zeros_small = jnp.zeros((ZW,), jnp.float32)

    p1 = _build_p1(rows, cols, zeros_small).reshape(NP, NP)
    p2 = _matmul(p1, p1)
    p3 = _matmul(p1, p2)
    p4 = _matmul(p2, p2)
    out8 = _diag_all(p1, p2, p3, p4)
    return out8.T[:N].astype(x.dtype)
